# Initial kernel scaffold; baseline (speedup 1.0000x reference)
import jax
import jax.numpy as jnp
from jax.experimental import pallas as pl


def kernel(spikes):
    raise NotImplementedError("write your pallas kernel here")


# speedup vs baseline: 8.6668x; 8.6668x over previous
import functools

import jax
import jax.numpy as jnp
from jax.experimental import pallas as pl

_TEMPERATURE = 0.2
_THR = 0.5


def _wta_kernel(x_ref, idx_ref, w_ref, gated_ref):
    x = x_ref[0]
    L, K = x.shape
    s = x > _THR
    flat = jax.lax.broadcasted_iota(jnp.int32, (L, K), 0) * K + \
        jax.lax.broadcasted_iota(jnp.int32, (L, K), 1)
    big = jnp.int32(L * K)
    m = jnp.min(jnp.where(s, flat, big))
    has_any = m < big
    k_star = jnp.remainder(m, K)
    total = jnp.sum(x, axis=0, keepdims=True)
    kiota = jax.lax.broadcasted_iota(jnp.int32, (1, K), 1)
    maxv = jnp.max(total)
    k_fb = jnp.min(jnp.where(total == maxv, kiota, jnp.int32(K)))
    idx = jnp.where(has_any, k_star, k_fb)
    w = (kiota == idx).astype(x.dtype)
    idx_ref[...] = jnp.broadcast_to(idx, (1, 1, 1))
    w_ref[0] = w
    gated_ref[0] = x * w


@jax.jit
def kernel(spikes):
    B, L, K = spikes.shape
    idx3, w3, gated = pl.pallas_call(
        _wta_kernel,
        grid=(B,),
        in_specs=[pl.BlockSpec((1, L, K), lambda b: (b, 0, 0))],
        out_specs=[
            pl.BlockSpec((1, 1, 1), lambda b: (b, 0, 0)),
            pl.BlockSpec((1, 1, K), lambda b: (b, 0, 0)),
            pl.BlockSpec((1, L, K), lambda b: (b, 0, 0)),
        ],
        out_shape=[
            jax.ShapeDtypeStruct((B, 1, 1), jnp.int32),
            jax.ShapeDtypeStruct((B, 1, K), spikes.dtype),
            jax.ShapeDtypeStruct((B, L, K), spikes.dtype),
        ],
    )(spikes)
    return idx3.reshape(B), w3.reshape(B, K), gated
